# baseline (speedup 1.0000x reference)
import functools

import jax
import jax.numpy as jnp
from jax import lax
from jax.experimental import pallas as pl
from jax.experimental.pallas import tpu as pltpu
from jax.experimental.pallas import tpu_sc as plsc

_NUM_CLASSES = 80
_L_COORD = 0.05
_L_CONF = 1.0
_L_CLS = 0.5
_B, _H, _W, _A = 16, 64, 64, 3
_N = 64
_HW = _H * _W
_CH = _A * (5 + _NUM_CLASSES)
_ROWS = _B * _HW * _A
_NB = _B * _N
_BPT = _NB // 32


def _sc_route_body(bb_hbm, anc_hbm, cells_hbm, best_hbm,
                   xbuf, ybuf, wbuf, hbuf, abuf, cbuf, obuf):
    wid = lax.axis_index("s") * 2 + lax.axis_index("c")
    base = wid * _BPT
    pltpu.sync_copy(bb_hbm.at[pl.ds(0 * _NB + base, _BPT)], xbuf)
    pltpu.sync_copy(bb_hbm.at[pl.ds(1 * _NB + base, _BPT)], ybuf)
    pltpu.sync_copy(bb_hbm.at[pl.ds(2 * _NB + base, _BPT)], wbuf)
    pltpu.sync_copy(bb_hbm.at[pl.ds(3 * _NB + base, _BPT)], hbuf)
    pltpu.sync_copy(anc_hbm, abuf)
    for v in range(_BPT // 16):
        x = xbuf[pl.ds(v * 16, 16)]
        y = ybuf[pl.ds(v * 16, 16)]
        w = wbuf[pl.ds(v * 16, 16)]
        h = hbuf[pl.ds(v * 16, 16)]
        cx = x + w * 0.5
        cy = y + h * 0.5
        fx = (cx * jnp.float32(_W)).astype(jnp.int32)
        fy = (cy * jnp.float32(_H)).astype(jnp.int32)
        cbuf[pl.ds(v * 16, 16)] = fy * _W + fx
        ax0 = abuf[pl.ds(0, 16)]
        ay0 = abuf[pl.ds(16, 16)]
        ax1 = abuf[pl.ds(32, 16)]
        ay1 = abuf[pl.ds(48, 16)]
        ax2 = abuf[pl.ds(64, 16)]
        ay2 = abuf[pl.ds(80, 16)]
        q0 = jnp.maximum(w / ax0, ax0 / w) * jnp.maximum(h / ay0, ay0 / h)
        q1 = jnp.maximum(w / ax1, ax1 / w) * jnp.maximum(h / ay1, ay1 / h)
        q2 = jnp.maximum(w / ax2, ax2 / w) * jnp.maximum(h / ay2, ay2 / h)
        s01 = jnp.minimum(jnp.sign(q1 - q0) + 1.0, 1.0)
        s02 = jnp.minimum(jnp.sign(q2 - q0) + 1.0, 1.0)
        s12 = jnp.minimum(jnp.sign(q2 - q1) + 1.0, 1.0)
        is0 = s01 * s02
        best_f = (1.0 - is0) * (s12 + 2.0 * (1.0 - s12))
        obuf[pl.ds(v * 16, 16)] = best_f.astype(jnp.int32)
    pltpu.sync_copy(cbuf, cells_hbm.at[pl.ds(base, _BPT)])
    pltpu.sync_copy(obuf, best_hbm.at[pl.ds(base, _BPT)])


def _sc_route(bb_flat, anc_pad):
    mesh = plsc.VectorSubcoreMesh(core_axis_name="c", subcore_axis_name="s")
    fn = functools.partial(
        pl.kernel,
        mesh=mesh,
        out_type=[jax.ShapeDtypeStruct((_NB,), jnp.int32),
                  jax.ShapeDtypeStruct((_NB,), jnp.int32)],
        scratch_types=[pltpu.VMEM((_BPT,), jnp.float32),
                       pltpu.VMEM((_BPT,), jnp.float32),
                       pltpu.VMEM((_BPT,), jnp.float32),
                       pltpu.VMEM((_BPT,), jnp.float32),
                       pltpu.VMEM((96,), jnp.float32),
                       pltpu.VMEM((_BPT,), jnp.int32),
                       pltpu.VMEM((_BPT,), jnp.int32)],
    )(_sc_route_body)
    return fn(bb_flat, anc_pad)


def _softplus(x):
    return jnp.maximum(x, 0.0) + jnp.log1p(jnp.exp(-jnp.abs(x)))


def _image_terms(p, bb, lab, cells, best, anc):
    eps = 1e-8
    cell = 1.0 / jnp.float32(_W)
    cxy = bb[:, 0:2] + bb[:, 2:4] * 0.5
    cij_fx = (cells % _W).astype(jnp.float32)
    cij_fy = (cells // _W).astype(jnp.float32)
    cij_f = jnp.concatenate([cij_fx, cij_fy], axis=1)
    frac = (cxy - cij_f * cell) / cell + eps
    txy = -jnp.log(1.0 / frac - 1.0)

    log_wh = jnp.log(bb[:, 2:4])
    log_anc = jnp.log(anc)
    twh0 = log_wh - log_anc[0:1, :]
    twh1 = log_wh - log_anc[1:2, :]
    twh2 = log_wh - log_anc[2:3, :]
    m0 = (best == 0).astype(jnp.float32)
    m1 = (best == 1).astype(jnp.float32)
    m2 = (best == 2).astype(jnp.float32)
    sS0 = jnp.max(m0)
    sS1 = jnp.max(m1)
    sS2 = jnp.max(m2)
    n_obj_i = jnp.float32(_N) * (sS0 + sS1 + sS2)

    pos = lax.broadcasted_iota(jnp.int32, (_N, _HW), 1)
    onehot = (cells == pos).astype(jnp.float32)
    g = jnp.dot(onehot, p, preferred_element_type=jnp.float32)

    coord = jnp.float32(0.0)
    objx = jnp.float32(0.0)
    for a, (twh_a, ss_a) in enumerate(((twh0, sS0), (twh1, sS1), (twh2, sS2))):
        base = a * (5 + _NUM_CLASSES)
        d_xy = g[:, base:base + 2] - txy
        d_wh = g[:, base + 2:base + 4] - twh_a
        coord += ss_a * (jnp.sum(d_xy * d_xy) + jnp.sum(d_wh * d_wh))
        objx += ss_a * jnp.sum(g[:, base + 4:base + 5])

    conf_cols = jnp.concatenate(
        [p[:, 4:5].reshape(32, 128),
         p[:, 89:90].reshape(32, 128),
         p[:, 174:175].reshape(32, 128)], axis=0)
    sp = jnp.sum(_softplus(conf_cols))

    z = m0 * g[:, 5:85] + m1 * g[:, 90:170] + m2 * g[:, 175:255]
    zmax = jnp.max(z, axis=1, keepdims=True)
    lse = zmax + jnp.log(jnp.sum(jnp.exp(z - zmax), axis=1, keepdims=True))
    cls_iota = lax.broadcasted_iota(jnp.int32, (_N, _NUM_CLASSES), 1)
    onehot_lab = (lab == cls_iota).astype(jnp.float32)
    z_lab = jnp.sum(onehot_lab * z, axis=1, keepdims=True)
    ce = jnp.sum(lse - z_lab)

    return coord, sp, objx, ce, n_obj_i


def _loss_body(pred_ref, bb_ref, lab_ref, cells_ref, best_ref, anc_ref, out_ref):
    i = pl.program_id(0)

    @pl.when(i == 0)
    def _init():
        out_ref[...] = jnp.zeros_like(out_ref)

    anc = anc_ref[...]
    coord, sp, objx, ce, n_obj = _image_terms(
        pred_ref[0], bb_ref[0], lab_ref[0], cells_ref[0], best_ref[0], anc)

    row = lax.broadcasted_iota(jnp.int32, (8, 128), 0)
    lane = lax.broadcasted_iota(jnp.int32, (8, 128), 1)
    r0 = row == 0
    contrib = jnp.where(jnp.logical_and(r0, lane == 0), coord, 0.0)
    contrib += jnp.where(jnp.logical_and(r0, lane == 1), sp, 0.0)
    contrib += jnp.where(jnp.logical_and(r0, lane == 2), objx, 0.0)
    contrib += jnp.where(jnp.logical_and(r0, lane == 3), ce, 0.0)
    contrib += jnp.where(jnp.logical_and(r0, lane == 4), n_obj, 0.0)
    out_ref[...] += contrib


def kernel(pred, bboxes, labels, anchors):
    pred_r = pred.reshape(_B, _HW, _CH)
    lab_r = labels.reshape(_B, _N, 1).astype(jnp.int32)

    bb_t = bboxes.reshape(_NB, 4).T.reshape(-1)
    anc_b = jnp.broadcast_to(anchors.reshape(-1)[:, None], (6, 16)).reshape(-1)
    cells, best = _sc_route(bb_t, anc_b)
    cells_r = cells.reshape(_B, _N, 1)
    best_r = best.reshape(_B, _N, 1)

    out = pl.pallas_call(
        _loss_body,
        grid=(_B,),
        in_specs=[
            pl.BlockSpec((1, _HW, _CH), lambda i: (i, 0, 0)),
            pl.BlockSpec((1, _N, 4), lambda i: (i, 0, 0)),
            pl.BlockSpec((1, _N, 1), lambda i: (i, 0, 0)),
            pl.BlockSpec((1, _N, 1), lambda i: (i, 0, 0)),
            pl.BlockSpec((1, _N, 1), lambda i: (i, 0, 0)),
            pl.BlockSpec((_A, 2), lambda i: (0, 0)),
        ],
        out_specs=pl.BlockSpec((8, 128), lambda i: (0, 0)),
        out_shape=jax.ShapeDtypeStruct((8, 128), jnp.float32),
    )(pred_r, bboxes, lab_r, cells_r, best_r, anchors)

    o = out[0]
    coord_sum, sp_sum, objx, ce_sum, n_obj = o[0], o[1], o[2], o[3], o[4]
    coord_loss = _L_COORD * coord_sum / (n_obj * 4.0)
    conf_loss = _L_CONF * (sp_sum - objx) / jnp.float32(_ROWS)
    class_loss = _L_CLS * ce_sum / n_obj
    loss = coord_loss + conf_loss + class_loss
    return (loss, coord_loss, conf_loss, class_loss)


# speedup vs baseline: 1.0216x; 1.0216x over previous
import functools

import jax
import jax.numpy as jnp
from jax import lax
from jax.experimental import pallas as pl
from jax.experimental.pallas import tpu as pltpu
from jax.experimental.pallas import tpu_sc as plsc

_NUM_CLASSES = 80
_L_COORD = 0.05
_L_CONF = 1.0
_L_CLS = 0.5
_B, _H, _W, _A = 16, 64, 64, 3
_N = 64
_HW = _H * _W
_CH = _A * (5 + _NUM_CLASSES)
_ROWS = _B * _HW * _A
_NB = _B * _N
_BPT = _NB // 32


def _sc_route_body(bb_hbm, anc_hbm, packed_hbm,
                   xbuf, ybuf, wbuf, hbuf, abuf, cbuf):
    wid = lax.axis_index("s") * 2 + lax.axis_index("c")
    base = wid * _BPT
    pltpu.sync_copy(bb_hbm.at[pl.ds(0 * _NB + base, _BPT)], xbuf)
    pltpu.sync_copy(bb_hbm.at[pl.ds(1 * _NB + base, _BPT)], ybuf)
    pltpu.sync_copy(bb_hbm.at[pl.ds(2 * _NB + base, _BPT)], wbuf)
    pltpu.sync_copy(bb_hbm.at[pl.ds(3 * _NB + base, _BPT)], hbuf)
    pltpu.sync_copy(anc_hbm, abuf)
    for v in range(_BPT // 16):
        x = xbuf[pl.ds(v * 16, 16)]
        y = ybuf[pl.ds(v * 16, 16)]
        w = wbuf[pl.ds(v * 16, 16)]
        h = hbuf[pl.ds(v * 16, 16)]
        cx = x + w * 0.5
        cy = y + h * 0.5
        fx = (cx * jnp.float32(_W)).astype(jnp.int32)
        fy = (cy * jnp.float32(_H)).astype(jnp.int32)
        ax0 = abuf[pl.ds(0, 16)]
        ay0 = abuf[pl.ds(16, 16)]
        ax1 = abuf[pl.ds(32, 16)]
        ay1 = abuf[pl.ds(48, 16)]
        ax2 = abuf[pl.ds(64, 16)]
        ay2 = abuf[pl.ds(80, 16)]
        q0 = jnp.maximum(w / ax0, ax0 / w) * jnp.maximum(h / ay0, ay0 / h)
        q1 = jnp.maximum(w / ax1, ax1 / w) * jnp.maximum(h / ay1, ay1 / h)
        q2 = jnp.maximum(w / ax2, ax2 / w) * jnp.maximum(h / ay2, ay2 / h)
        s01 = jnp.minimum(jnp.sign(q1 - q0) + 1.0, 1.0)
        s02 = jnp.minimum(jnp.sign(q2 - q0) + 1.0, 1.0)
        s12 = jnp.minimum(jnp.sign(q2 - q1) + 1.0, 1.0)
        is0 = s01 * s02
        best_f = (1.0 - is0) * (s12 + 2.0 * (1.0 - s12))
        cbuf[pl.ds(v * 16, 16)] = (fy * _W + fx) + _HW * best_f.astype(jnp.int32)
    pltpu.sync_copy(cbuf, packed_hbm.at[pl.ds(base, _BPT)])


def _sc_route(bb_flat, anc_pad):
    mesh = plsc.VectorSubcoreMesh(core_axis_name="c", subcore_axis_name="s")
    fn = functools.partial(
        pl.kernel,
        mesh=mesh,
        out_type=jax.ShapeDtypeStruct((_NB,), jnp.int32),
        scratch_types=[pltpu.VMEM((_BPT,), jnp.float32),
                       pltpu.VMEM((_BPT,), jnp.float32),
                       pltpu.VMEM((_BPT,), jnp.float32),
                       pltpu.VMEM((_BPT,), jnp.float32),
                       pltpu.VMEM((96,), jnp.float32),
                       pltpu.VMEM((_BPT,), jnp.int32)],
    )(_sc_route_body)
    return fn(bb_flat, anc_pad)


def _softplus(x):
    return jnp.maximum(x, 0.0) + jnp.log1p(jnp.exp(-jnp.abs(x)))


def _image_terms(p, bb, lab, packed, anc):
    cells = packed % _HW
    best = packed // _HW
    eps = 1e-8
    cell = 1.0 / jnp.float32(_W)
    cxy = bb[:, 0:2] + bb[:, 2:4] * 0.5
    cij_fx = (cells % _W).astype(jnp.float32)
    cij_fy = (cells // _W).astype(jnp.float32)
    cij_f = jnp.concatenate([cij_fx, cij_fy], axis=1)
    frac = (cxy - cij_f * cell) / cell + eps
    txy = -jnp.log(1.0 / frac - 1.0)

    log_wh = jnp.log(bb[:, 2:4])
    log_anc = jnp.log(anc)
    twh0 = log_wh - log_anc[0:1, :]
    twh1 = log_wh - log_anc[1:2, :]
    twh2 = log_wh - log_anc[2:3, :]
    m0 = (best == 0).astype(jnp.float32)
    m1 = (best == 1).astype(jnp.float32)
    m2 = (best == 2).astype(jnp.float32)
    sS0 = jnp.max(m0)
    sS1 = jnp.max(m1)
    sS2 = jnp.max(m2)
    n_obj_i = jnp.float32(_N) * (sS0 + sS1 + sS2)

    pos = lax.broadcasted_iota(jnp.int32, (_N, _HW), 1)
    onehot = (cells == pos).astype(jnp.float32)
    g = jnp.dot(onehot, p, preferred_element_type=jnp.float32)

    coord = jnp.float32(0.0)
    objx = jnp.float32(0.0)
    for a, (twh_a, ss_a) in enumerate(((twh0, sS0), (twh1, sS1), (twh2, sS2))):
        base = a * (5 + _NUM_CLASSES)
        d_xy = g[:, base:base + 2] - txy
        d_wh = g[:, base + 2:base + 4] - twh_a
        coord += ss_a * (jnp.sum(d_xy * d_xy) + jnp.sum(d_wh * d_wh))
        objx += ss_a * jnp.sum(g[:, base + 4:base + 5])

    conf_cols = jnp.concatenate(
        [p[:, 4:5].reshape(32, 128),
         p[:, 89:90].reshape(32, 128),
         p[:, 174:175].reshape(32, 128)], axis=0)
    sp = jnp.sum(_softplus(conf_cols))

    z = m0 * g[:, 5:85] + m1 * g[:, 90:170] + m2 * g[:, 175:255]
    zmax = jnp.max(z, axis=1, keepdims=True)
    lse = zmax + jnp.log(jnp.sum(jnp.exp(z - zmax), axis=1, keepdims=True))
    cls_iota = lax.broadcasted_iota(jnp.int32, (_N, _NUM_CLASSES), 1)
    onehot_lab = (lab == cls_iota).astype(jnp.float32)
    z_lab = jnp.sum(onehot_lab * z, axis=1, keepdims=True)
    ce = jnp.sum(lse - z_lab)

    return coord, sp, objx, ce, n_obj_i


def _loss_body(pred_ref, bb_ref, lab_ref, packed_ref, anc_ref, out_ref):
    i = pl.program_id(0)

    @pl.when(i == 0)
    def _init():
        out_ref[...] = jnp.zeros_like(out_ref)

    anc = anc_ref[...]
    coord, sp, objx, ce, n_obj = _image_terms(
        pred_ref[0], bb_ref[0], lab_ref[0], packed_ref[0], anc)

    row = lax.broadcasted_iota(jnp.int32, (8, 128), 0)
    lane = lax.broadcasted_iota(jnp.int32, (8, 128), 1)
    r0 = row == 0
    contrib = jnp.where(jnp.logical_and(r0, lane == 0), coord, 0.0)
    contrib += jnp.where(jnp.logical_and(r0, lane == 1), sp, 0.0)
    contrib += jnp.where(jnp.logical_and(r0, lane == 2), objx, 0.0)
    contrib += jnp.where(jnp.logical_and(r0, lane == 3), ce, 0.0)
    contrib += jnp.where(jnp.logical_and(r0, lane == 4), n_obj, 0.0)
    out_ref[...] += contrib


def kernel(pred, bboxes, labels, anchors):
    pred_r = pred.reshape(_B, _HW, _CH)
    lab_r = labels.reshape(_B, _N, 1).astype(jnp.int32)

    bb_t = bboxes.reshape(_NB, 4).T.reshape(-1)
    anc_b = jnp.broadcast_to(anchors.reshape(-1)[:, None], (6, 16)).reshape(-1)
    packed = _sc_route(bb_t, anc_b)
    packed_r = packed.reshape(_B, _N, 1)

    out = pl.pallas_call(
        _loss_body,
        grid=(_B,),
        in_specs=[
            pl.BlockSpec((1, _HW, _CH), lambda i: (i, 0, 0)),
            pl.BlockSpec((1, _N, 4), lambda i: (i, 0, 0)),
            pl.BlockSpec((1, _N, 1), lambda i: (i, 0, 0)),
            pl.BlockSpec((1, _N, 1), lambda i: (i, 0, 0)),
            pl.BlockSpec((_A, 2), lambda i: (0, 0)),
        ],
        out_specs=pl.BlockSpec((8, 128), lambda i: (0, 0)),
        out_shape=jax.ShapeDtypeStruct((8, 128), jnp.float32),
    )(pred_r, bboxes, lab_r, packed_r, anchors)

    o = out[0]
    coord_sum, sp_sum, objx, ce_sum, n_obj = o[0], o[1], o[2], o[3], o[4]
    coord_loss = _L_COORD * coord_sum / (n_obj * 4.0)
    conf_loss = _L_CONF * (sp_sum - objx) / jnp.float32(_ROWS)
    class_loss = _L_CLS * ce_sum / n_obj
    loss = coord_loss + conf_loss + class_loss
    return (loss, coord_loss, conf_loss, class_loss)
